# baseline (speedup 1.0000x reference)
import functools
import math

import jax
import jax.numpy as jnp
from jax import lax
from jax.experimental import pallas as pl
from jax.experimental.pallas import tpu as pltpu
from jax.experimental.pallas import tpu_sc as plsc

_SIZE = 1000
_PAD = 0
_SMOOTH = 0.1
_CONF = 1.0 - _SMOOTH
_EPS = _SMOOTH / (_SIZE - 2)
_ROW_CONST = _SMOOTH * math.log(_EPS) + _CONF * math.log(_CONF)

_BC = 2048
_SPLIT = 744
_LANES = 16


def _tc_body(xt_ref, t_ref, a_ref, n_ref):
    pid = pl.program_id(0)

    @pl.when(pid == 0)
    def _():
        a_ref[0, 0] = 0.0
        n_ref[0, 0] = 0.0

    xb = xt_ref[...]
    t = t_ref[...]
    mask = t != _PAD
    rows = lax.broadcasted_iota(jnp.int32, xb.shape, 0)
    w = jnp.where(rows == t, -_CONF, -_EPS)
    w = jnp.where((rows == _PAD) | (~mask), 0.0, w)
    a_ref[0, 0] += jnp.sum(xb * w)
    n_ref[0, 0] += jnp.sum(jnp.where(mask, 1.0, 0.0))


def _tc_weighted_sum(xt, t2d):
    n_cols = xt.shape[1]
    scalar_spec = pl.BlockSpec((1, 1), lambda i: (0, 0),
                               memory_space=pltpu.SMEM)
    return pl.pallas_call(
        _tc_body,
        grid=(n_cols // _BC,),
        in_specs=[
            pl.BlockSpec((_SPLIT, _BC), lambda i: (0, i)),
            pl.BlockSpec((1, _BC), lambda i: (0, i)),
        ],
        out_specs=[scalar_spec, scalar_spec],
        out_shape=[jax.ShapeDtypeStruct((1, 1), jnp.float32)] * 2,
    )(xt, t2d)


def _sc_weighted_sum(xt, tgt):
    info = plsc.get_sparse_core_info()
    nc, ns = info.num_cores, info.num_subcores
    nw = nc * ns
    n_cols = xt.shape[1]
    rows_sc = _SIZE - _SPLIT
    rpw = rows_sc // nw
    rchunk = 2

    @functools.partial(
        pl.kernel,
        mesh=plsc.VectorSubcoreMesh(core_axis_name="c", subcore_axis_name="s"),
        out_type=jax.ShapeDtypeStruct((nw, _LANES), jnp.float32),
        scratch_types=[
            pltpu.VMEM((n_cols,), jnp.int32),
            pltpu.VMEM((rchunk, n_cols), jnp.float32),
            pltpu.VMEM((_LANES,), jnp.float32),
        ],
    )
    def k(xt_hbm, t_hbm, out_hbm, t_v, buf_v, res_v):
        wid = lax.axis_index("s") * nc + lax.axis_index("c")
        row0 = _SPLIT + wid * rpw
        pltpu.sync_copy(t_hbm, t_v)
        acc = jnp.zeros((_LANES,), jnp.float32)
        for it in range(rpw // rchunk):
            pltpu.sync_copy(xt_hbm.at[pl.ds(row0 + it * rchunk, rchunk)],
                            buf_v)
            r0 = row0 + it * rchunk
            r1 = r0 + 1

            def body(j, acc):
                sl = pl.ds(j * _LANES, _LANES)
                t16 = t_v[sl]
                x0 = buf_v[0, sl]
                x1 = buf_v[1, sl]
                w0 = jnp.where(t16 == r0, -_CONF, -_EPS)
                w1 = jnp.where(t16 == r1, -_CONF, -_EPS)
                contrib = x0 * w0 + x1 * w1
                return acc + jnp.where(t16 != _PAD, contrib, 0.0)

            acc = lax.fori_loop(0, n_cols // _LANES, body, acc)
        res_v[...] = acc
        pltpu.sync_copy(res_v, out_hbm.at[wid])

    return k(xt, tgt)


def kernel(x, target):
    n_rows = x.shape[0]
    t32 = target.astype(jnp.int32)
    xt = x.T
    sc_parts = _sc_weighted_sum(xt, t32)
    a, n = _tc_weighted_sum(xt, t32.reshape(1, n_rows))
    total = n[0, 0] * _ROW_CONST + a[0, 0] + jnp.sum(sc_parts)
    return total.astype(jnp.float32)


# speedup vs baseline: 1.9520x; 1.9520x over previous
import math

import jax
import jax.numpy as jnp
from jax import lax
from jax.experimental import pallas as pl
from jax.experimental.pallas import tpu as pltpu

_SIZE = 1000
_PAD = 0
_SMOOTH = 0.1
_CONF = 1.0 - _SMOOTH
_EPS = _SMOOTH / (_SIZE - 2)
_ROW_CONST = _SMOOTH * math.log(_EPS) + _CONF * math.log(_CONF)

_BC = 2048


def _tc_body(xt_ref, t_ref, a_ref, n_ref):
    pid = pl.program_id(0)

    @pl.when(pid == 0)
    def _():
        a_ref[0, 0] = 0.0
        n_ref[0, 0] = 0.0

    xb = xt_ref[...]
    t = t_ref[...]
    mask = t != _PAD
    rows = lax.broadcasted_iota(jnp.int32, xb.shape, 0)
    w = jnp.where(rows == t, -_CONF, -_EPS)
    w = jnp.where((rows == _PAD) | (~mask), 0.0, w)
    a_ref[0, 0] += jnp.sum(xb * w)
    n_ref[0, 0] += jnp.sum(jnp.where(mask, 1.0, 0.0))


def _tc_weighted_sum(xt, t2d):
    n_cols = xt.shape[1]
    scalar_spec = pl.BlockSpec((1, 1), lambda i: (0, 0),
                               memory_space=pltpu.SMEM)
    return pl.pallas_call(
        _tc_body,
        grid=(n_cols // _BC,),
        in_specs=[
            pl.BlockSpec((_SIZE, _BC), lambda i: (0, i)),
            pl.BlockSpec((1, _BC), lambda i: (0, i)),
        ],
        out_specs=[scalar_spec, scalar_spec],
        out_shape=[jax.ShapeDtypeStruct((1, 1), jnp.float32)] * 2,
    )(xt, t2d)


def kernel(x, target):
    n_rows = x.shape[0]
    t32 = target.astype(jnp.int32)
    a, n = _tc_weighted_sum(x.T, t32.reshape(1, n_rows))
    total = n[0, 0] * _ROW_CONST + a[0, 0]
    return total.astype(jnp.float32)
